# baseline (speedup 1.0000x reference)
import functools

import jax
import jax.numpy as jnp
import numpy as np
from jax import lax
from jax.experimental import pallas as pl
from jax.experimental.pallas import tpu as pltpu
from jax.experimental.pallas import tpu_sc as plsc

N = 10000
NPAD = 10240
NE = 320000
NEPAD = 327680
IN_DIM = 128
C = 64
NC, NS, LANES = 2, 16, 16
NW = NC * NS
EPW = NEPAD // NW
CH = 128
NCHUNK = EPW // CH
ZROWS = NPAD // NS
OROWS = 1000

f32 = jnp.float32
i32 = jnp.int32
_Z = np.int32(0)


def _sc_body(first, src_h, dst_h, ae_h, asrc_h, adst_h, h_h, z64_h, z16_h,
             *rest):
    assert first
    (outp_h, outd_h, outs_h,
     acc_p, acc_d, acc_s,
     asrc_v, adst_v, srcv, dstv, rows_a, rows_b, den2, idxb, sbuf,
     sem_a, sem_b) = rest

    cid = lax.axis_index("c")
    sid = lax.axis_index("s")
    wid = sid * NC + cid
    tid = sid

    pltpu.sync_copy(asrc_h, asrc_v)
    pltpu.sync_copy(adst_h, adst_v)
    pltpu.sync_copy(src_h.at[wid], srcv)
    pltpu.sync_copy(dst_h.at[wid], dstv)

    pltpu.sync_copy(z64_h, acc_p.at[pl.ds(tid * ZROWS, ZROWS), :])
    pltpu.sync_copy(z16_h.at[pl.ds(0, ZROWS // NS), :],
                    acc_d.at[pl.ds(tid * (ZROWS // NS), ZROWS // NS), :])
    if first:
        pltpu.sync_copy(z16_h, acc_s.at[pl.ds(tid * ZROWS, ZROWS), :])
    pltpu.sync_copy(z16_h, den2)
    for j in range(ZROWS // CH):
        for g in range(CH // LANES):
            idxb[j, pl.ds(g * LANES, LANES)] = (
                lax.iota(i32, LANES) + (j * CH + g * LANES))
    plsc.subcore_barrier()

    def issue_gather(c, rows, sem):
        pltpu.async_copy(h_h.at[srcv.at[c]], rows, sem)

    def wait_gather(c, rows, sem):
        pltpu.make_async_copy(h_h.at[srcv.at[c]], rows, sem).wait()

    def do_chunk(c, rows):
        if first:
            pltpu.sync_copy(ae_h.at[wid, c], sbuf)

        def grp(g, carry2):
            sl = pl.ds(g * np.int32(LANES), LANES)
            s16 = srcv[c, sl]
            d16 = dstv[c, sl]
            a1 = plsc.load_gather(asrc_v, [s16])
            a2 = plsc.load_gather(adst_v, [d16])
            e16 = lax.iota(i32, LANES) + g * np.int32(LANES)
            if first:
                ae16 = plsc.load_gather(sbuf, [e16, jnp.full((LANES,), 5, i32)])
            else:
                ae16 = aev[c, sl]
            t = a1 + a2 + ae16
            t = jnp.maximum(t, f32(0.2) * t)
            p16 = jnp.exp(t)
            plsc.addupdate_scatter(
                den2, [lax.shift_right_logical(d16, 4),
                       lax.bitwise_and(d16, jnp.full((LANES,), 15, i32))],
                p16)
            for cc in range(C):
                ci = jnp.full((LANES,), cc, i32)
                rv = plsc.load_gather(rows, [e16, ci])
                plsc.store_scatter(rows, [e16, ci], rv * p16)
            return carry2

        lax.fori_loop(np.int32(0), np.int32(CH // LANES), grp, np.int32(0))
        pltpu.sync_copy(rows, acc_p.at[dstv.at[c]], add=True)
        if first:
            pltpu.sync_copy(sbuf, acc_s.at[dstv.at[c]], add=True)

    issue_gather(np.int32(0), rows_a, sem_a)

    def pair(i, carry):
        c0 = i * np.int32(2)
        c1 = c0 + np.int32(1)
        issue_gather(c1, rows_b, sem_b)
        wait_gather(c0, rows_a, sem_a)
        do_chunk(c0, rows_a)

        @pl.when(i < np.int32(NCHUNK // 2 - 1))
        def _prefetch():
            issue_gather(c1 + np.int32(1), rows_a, sem_a)

        wait_gather(c1, rows_b, sem_b)
        do_chunk(c1, rows_b)
        return carry

    lax.fori_loop(np.int32(0), np.int32(NCHUNK // 2), pair, np.int32(0))

    for j in range(ZROWS // CH):
        pltpu.sync_copy(den2.at[pl.ds(j * CH, CH), :],
                        acc_d.at[idxb.at[j]], add=True)
    plsc.subcore_barrier()

    @pl.when(tid < N // OROWS)
    def _copyout():
        r0 = tid * np.int32(OROWS)
        pltpu.sync_copy(acc_p.at[pl.ds(r0, OROWS), :],
                        outp_h.at[cid, pl.ds(r0, OROWS), :])
        if first:
            pltpu.sync_copy(acc_s.at[pl.ds(r0, OROWS), :],
                            outs_h.at[cid, pl.ds(r0, OROWS), :])

    @pl.when(tid == np.int32(0))
    def _copyden():
        pltpu.sync_copy(acc_d, outd_h.at[cid])


CHR = 256
NCHR = EPW // CHR


def _sc_rest_body(src_h, dst_h, ae_h, asrc_h, adst_h, h_h, z64_h, z16_h,
                  outp_h, outd_h,
                  acc_p, acc_d,
                  asrc_v, adst_v, srcv, dstv, rows_a, rows_b, aeb_a, aeb_b,
                  den2, idxb, sem_a, sem_b):
    cid = lax.axis_index("c")
    sid = lax.axis_index("s")
    wid = sid * NC + cid
    tid = sid

    pltpu.sync_copy(asrc_h, asrc_v)
    pltpu.sync_copy(adst_h, adst_v)
    pltpu.sync_copy(src_h.at[wid], srcv)
    pltpu.sync_copy(dst_h.at[wid], dstv)

    pltpu.sync_copy(z64_h, acc_p.at[pl.ds(tid * ZROWS, ZROWS), :])
    pltpu.sync_copy(z16_h.at[pl.ds(0, ZROWS // NS), :],
                    acc_d.at[pl.ds(tid * (ZROWS // NS), ZROWS // NS), :])
    pltpu.sync_copy(z16_h, den2)
    for j in range(ZROWS // CH):
        for g in range(CH // LANES):
            idxb[j, pl.ds(g * LANES, LANES)] = (
                lax.iota(i32, LANES) + (j * CH + g * LANES))
    plsc.subcore_barrier()

    def issue_gather(c, rows, aeb, sem):
        pltpu.async_copy(h_h.at[srcv.at[c, pl.ds(0, CH)]],
                         rows.at[pl.ds(0, CH), :], sem)
        pltpu.async_copy(h_h.at[srcv.at[c, pl.ds(CH, CH)]],
                         rows.at[pl.ds(CH, CH), :], sem)
        pltpu.async_copy(ae_h.at[wid, c], aeb, sem)

    def wait_gather(c, rows, aeb, sem):
        pltpu.make_async_copy(h_h.at[srcv.at[c, pl.ds(0, CH)]],
                              rows.at[pl.ds(0, CH), :], sem).wait()
        pltpu.make_async_copy(h_h.at[srcv.at[c, pl.ds(CH, CH)]],
                              rows.at[pl.ds(CH, CH), :], sem).wait()
        pltpu.make_async_copy(ae_h.at[wid, c], aeb, sem).wait()

    def do_chunk(c, rows, aeb):
        for k in range(CHR // CH):
            def grp(g, carry2):
                off = g * np.int32(LANES) + np.int32(k * CH)
                s16 = srcv[c, pl.ds(off, LANES)]
                d16 = dstv[c * np.int32(2) + np.int32(k), pl.ds(g * np.int32(LANES), LANES)]
                a1 = plsc.load_gather(asrc_v, [s16])
                a2 = plsc.load_gather(adst_v, [d16])
                ae16 = aeb[pl.ds(off, LANES)]
                t = a1 + a2 + ae16
                t = jnp.maximum(t, f32(0.2) * t)
                p16 = jnp.exp(t)
                plsc.addupdate_scatter(
                    den2, [lax.shift_right_logical(d16, 4),
                           lax.bitwise_and(d16, jnp.full((LANES,), 15, i32))],
                    p16)
                e16 = lax.iota(i32, LANES) + off
                for cc in range(C):
                    ci = jnp.full((LANES,), cc, i32)
                    rv = plsc.load_gather(rows, [e16, ci])
                    plsc.store_scatter(rows, [e16, ci], rv * p16)
                return carry2

            lax.fori_loop(np.int32(0), np.int32(CH // LANES), grp, np.int32(0))
            pltpu.sync_copy(rows.at[pl.ds(k * CH, CH), :],
                            acc_p.at[dstv.at[c * np.int32(2) + np.int32(k)]],
                            add=True)

    issue_gather(np.int32(0), rows_a, aeb_a, sem_a)

    def pair(i, carry):
        c0 = i * np.int32(2)
        c1 = c0 + np.int32(1)
        issue_gather(c1, rows_b, aeb_b, sem_b)
        wait_gather(c0, rows_a, aeb_a, sem_a)
        do_chunk(c0, rows_a, aeb_a)

        @pl.when(i < np.int32(NCHR // 2 - 1))
        def _prefetch():
            issue_gather(c1 + np.int32(1), rows_a, aeb_a, sem_a)

        wait_gather(c1, rows_b, aeb_b, sem_b)
        do_chunk(c1, rows_b, aeb_b)
        return carry

    lax.fori_loop(np.int32(0), np.int32(NCHR // 2), pair, np.int32(0))

    for j in range(ZROWS // CH):
        pltpu.sync_copy(den2.at[pl.ds(j * CH, CH), :],
                        acc_d.at[idxb.at[j]], add=True)
    plsc.subcore_barrier()

    @pl.when(tid < N // OROWS)
    def _copyout():
        r0 = tid * np.int32(OROWS)
        pltpu.sync_copy(acc_p.at[pl.ds(r0, OROWS), :],
                        outp_h.at[cid, pl.ds(r0, OROWS), :])

    @pl.when(tid == np.int32(0))
    def _copyden():
        pltpu.sync_copy(acc_d, outd_h.at[cid])


def _make_sc_rest():
    mesh = plsc.VectorSubcoreMesh(core_axis_name="c", subcore_axis_name="s")
    return functools.partial(
        pl.kernel,
        out_type=[
            jax.ShapeDtypeStruct((NC, N, C), f32),
            jax.ShapeDtypeStruct((NC, ZROWS, 16), f32),
        ],
        mesh=mesh,
        scratch_types=[
            pltpu.VMEM_SHARED((NPAD, C), f32),
            pltpu.VMEM_SHARED((ZROWS, 16), f32),
            pltpu.VMEM((NPAD,), f32),
            pltpu.VMEM((NPAD,), f32),
            pltpu.VMEM((NCHR, CHR), i32),
            pltpu.VMEM((2 * NCHR, CH), i32),
            pltpu.VMEM((CHR, C), f32),
            pltpu.VMEM((CHR, C), f32),
            pltpu.VMEM((CHR,), f32),
            pltpu.VMEM((CHR,), f32),
            pltpu.VMEM((ZROWS, 16), f32),
            pltpu.VMEM((ZROWS // CH, CH), i32),
            pltpu.SemaphoreType.DMA,
            pltpu.SemaphoreType.DMA,
        ],
        compiler_params=pltpu.CompilerParams(needs_layout_passes=False,
                                             use_tc_tiling_on_sc=False),
    )(_sc_rest_body)


def _make_sc(first):
    out_type = [
        jax.ShapeDtypeStruct((NC, N, C), f32),
        jax.ShapeDtypeStruct((NC, ZROWS, 16), f32),
    ]
    scratch = [
        pltpu.VMEM_SHARED((NPAD, C), f32),
        pltpu.VMEM_SHARED((ZROWS, 16), f32),
    ]
    tile_scr = [
        pltpu.VMEM((NPAD,), f32),
        pltpu.VMEM((NPAD,), f32),
        pltpu.VMEM((NCHUNK, CH), i32),
        pltpu.VMEM((NCHUNK, CH), i32),
    ]
    if not first:
        tile_scr.append(pltpu.VMEM((NCHUNK, CH), f32))
    tile_scr += [
        pltpu.VMEM((CH, C), f32),
        pltpu.VMEM((CH, C), f32),
        pltpu.VMEM((ZROWS, 16), f32),
        pltpu.VMEM((ZROWS // CH, CH), i32),
    ]
    if first:
        out_type.append(jax.ShapeDtypeStruct((NC, N, 16), f32))
        scratch.append(pltpu.VMEM_SHARED((NPAD, 16), f32))
        tile_scr.append(pltpu.VMEM((CH, 16), f32))
    mesh = plsc.VectorSubcoreMesh(core_axis_name="c", subcore_axis_name="s")
    return functools.partial(
        pl.kernel,
        out_type=out_type,
        mesh=mesh,
        scratch_types=(scratch + tile_scr
                       + [pltpu.SemaphoreType.DMA] * 2),
        compiler_params=pltpu.CompilerParams(needs_layout_passes=False,
                                             use_tc_tiling_on_sc=False),
    )(functools.partial(_sc_body, first))


_sc_first = _make_sc(True)
_sc_rest = _make_sc_rest()


_BE = 8192


def _ae_body(ea_ref, w_ref, out_ref, sr_ref):
    ea = ea_ref[...]
    ae = jnp.dot(ea, w_ref[...], preferred_element_type=f32)
    out_ref[...] = ae
    sr_ref[...] = jnp.concatenate(
        [ea, jnp.ones((_BE, 1), f32), ae[:, 0:1], jnp.zeros((_BE, 10), f32)],
        axis=1)


def _ae_all(ea_p, wecat_t):
    return pl.pallas_call(
        _ae_body,
        grid=(NEPAD // _BE,),
        in_specs=[
            pl.BlockSpec((_BE, 4), lambda i: (i, _Z)),
            pl.BlockSpec((4, 4), lambda i: (_Z, _Z)),
        ],
        out_specs=[
            pl.BlockSpec((_BE, 4), lambda i: (i, _Z)),
            pl.BlockSpec((_BE, 16), lambda i: (i, _Z)),
        ],
        out_shape=[
            jax.ShapeDtypeStruct((NEPAD, 4), f32),
            jax.ShapeDtypeStruct((NEPAD, 16), f32),
        ],
    )(ea_p, wecat_t)


_BN = 2000


def _prep_body(x_ref, w_ref, att_ref, h_ref, av_ref):
    h = jnp.dot(x_ref[...], w_ref[...], preferred_element_type=f32)
    h_ref[...] = h
    a1 = jnp.sum(h * att_ref[0:1, :], axis=1, keepdims=True)
    a2 = jnp.sum(h * att_ref[1:2, :], axis=1, keepdims=True)
    av_ref[...] = jnp.concatenate([a1, a2], axis=1)


def _prep1(x, w, attv):
    return pl.pallas_call(
        _prep_body,
        grid=(N // _BN,),
        in_specs=[
            pl.BlockSpec((_BN, IN_DIM), lambda i: (i, _Z)),
            pl.BlockSpec((IN_DIM, C), lambda i: (_Z, _Z)),
            pl.BlockSpec((2, C), lambda i: (_Z, _Z)),
        ],
        out_specs=[
            pl.BlockSpec((_BN, C), lambda i: (i, _Z)),
            pl.BlockSpec((_BN, 2), lambda i: (i, _Z)),
        ],
        out_shape=[
            jax.ShapeDtypeStruct((N, C), f32),
            jax.ShapeDtypeStruct((N, 2), f32),
        ],
    )(x, w, attv)


def _combine_body(mode, p0, p1, d0, d1, s0, s1, h_ref, av_ref, wec, bias,
                  scale, shift, wn, attn_or_b, xo_ref, o1_ref, o2_ref=None):
    num = p0[...] + p1[...]
    den = d0[:, 0:1] + d1[:, 0:1]
    s = s0[...] + s1[...]
    mean_ea = s[:, 0:4] / jnp.maximum(s[:, 4:5], f32(1.0))
    ae_self = jnp.sum(mean_ea * wec[0:1, 0:4], axis=1, keepdims=True)
    ts = av_ref[:, 0:1] + av_ref[:, 1:2] + ae_self
    ts = jnp.maximum(ts, f32(0.2) * ts)
    ps = jnp.exp(ts)
    xo = (num + ps * h_ref[...]) / (den + ps + f32(1e-16)) + bias[0:1, :]
    xo = jnp.maximum(xo * scale[0:1, :] + shift[0:1, :], f32(0.0))
    xo_ref[...] = xo
    if mode < 2:
        hn = jnp.dot(xo, wn[...], preferred_element_type=f32)
        o1_ref[...] = hn
        a1 = jnp.sum(hn * attn_or_b[0:1, :], axis=1, keepdims=True)
        a2 = jnp.sum(hn * attn_or_b[1:2, :], axis=1, keepdims=True)
        o2_ref[...] = jnp.concatenate([a1, a2], axis=1)
    else:
        o1_ref[...] = (jnp.dot(xo, wn[...], preferred_element_type=f32)
                       + attn_or_b[0:1, :])


def _combine(mode, p, d, s, h, av, wec, bias, scale, shift, wn, attn_or_b):
    bcast = lambda i: (_Z, _Z)
    row64 = pl.BlockSpec((_BN, C), lambda i: (i, _Z))
    row16 = pl.BlockSpec((_BN, 16), lambda i: (i, _Z))
    row1 = pl.BlockSpec((_BN, 1), lambda i: (i, _Z))
    in_specs = [
        row64, row64, row1, row1, row16, row16, row64,
        pl.BlockSpec((_BN, 2), lambda i: (i, _Z)),
        pl.BlockSpec((1, 16), bcast),
        pl.BlockSpec((1, C), bcast),
        pl.BlockSpec((1, C), bcast),
        pl.BlockSpec((1, C), bcast),
    ]
    if mode < 2:
        in_specs += [pl.BlockSpec((C, C), bcast), pl.BlockSpec((2, C), bcast)]
        out_specs = [row64, row64, pl.BlockSpec((_BN, 2), lambda i: (i, _Z))]
        out_shape = [
            jax.ShapeDtypeStruct((N, C), f32),
            jax.ShapeDtypeStruct((N, C), f32),
            jax.ShapeDtypeStruct((N, 2), f32),
        ]
    else:
        in_specs += [pl.BlockSpec((C, 3 * C), bcast),
                     pl.BlockSpec((1, 3 * C), bcast)]
        out_specs = [row64, pl.BlockSpec((_BN, 3 * C), lambda i: (i, _Z))]
        out_shape = [
            jax.ShapeDtypeStruct((N, C), f32),
            jax.ShapeDtypeStruct((N, 3 * C), f32),
        ]
    return pl.pallas_call(
        functools.partial(_combine_body, mode),
        grid=(N // _BN,),
        in_specs=in_specs,
        out_specs=out_specs,
        out_shape=out_shape,
    )(p[0], p[1], d[0], d[1], s[0], s[1], h, av, wec, bias, scale, shift,
      wn, attn_or_b)


_BQ = 1000


def _mha_body(qkv_ref, qkvf_ref, x_ref, outw_ref, outb_ref, y_ref):
    q = qkv_ref[:, 0:C]
    nkb = N // _BQ

    qb16 = q.astype(jnp.bfloat16)

    def kblk(j, carry):
        num, den = carry
        jb = j * np.int32(_BQ)
        kb = qkvf_ref[pl.ds(jb, _BQ), C:2 * C].astype(jnp.bfloat16)
        vb = qkvf_ref[pl.ds(jb, _BQ), 2 * C:3 * C].astype(jnp.bfloat16)
        sc = lax.dot_general(qb16, kb, (((1,), (1,)), ((), ())),
                             preferred_element_type=f32) * f32(0.125)
        e = jnp.exp(sc)
        den = den + jnp.sum(e, axis=1, keepdims=True)
        num = num + jnp.dot(e.astype(jnp.bfloat16), vb,
                            preferred_element_type=f32)
        return num, den

    num0 = jnp.zeros((_BQ, C), f32)
    den0 = jnp.zeros((_BQ, 1), f32)
    num, den = lax.fori_loop(np.int32(0), np.int32(nkb), kblk, (num0, den0))
    y = num / den
    y_ref[...] = (jnp.dot(y, outw_ref[...], preferred_element_type=f32)
                  + outb_ref[0:1, :] + x_ref[...])


def _mha(qkv, x3, outw_t, outb):
    return pl.pallas_call(
        _mha_body,
        grid=(N // _BQ,),
        in_specs=[
            pl.BlockSpec((_BQ, 3 * C), lambda i: (i, _Z)),
            pl.BlockSpec((N, 3 * C), lambda i: (_Z, _Z)),
            pl.BlockSpec((_BQ, C), lambda i: (i, _Z)),
            pl.BlockSpec((C, C), lambda i: (_Z, _Z)),
            pl.BlockSpec((1, C), lambda i: (_Z, _Z)),
        ],
        out_specs=pl.BlockSpec((_BQ, C), lambda i: (i, _Z)),
        out_shape=jax.ShapeDtypeStruct((N, C), f32),
    )(qkv, qkv, x3, outw_t, outb)


def _head_body(y_ref, w1, b1, w2, b2, w3, b3, out_ref):
    y = y_ref[...]
    g = jnp.concatenate([jnp.mean(y, axis=0, keepdims=True),
                         jnp.max(y, axis=0, keepdims=True)], axis=1)
    h1 = jnp.maximum(jnp.dot(g, w1[...], preferred_element_type=f32)
                     + b1[0:1, :], f32(0.0))
    h2 = jnp.maximum(jnp.dot(h1, w2[...], preferred_element_type=f32)
                     + b2[0:1, :], f32(0.0))
    out_ref[...] = (jnp.dot(h2, w3[...], preferred_element_type=f32)
                    + b3[0:1, :])


def _head(y, w1, b1, w2, b2, w3, b3):
    return pl.pallas_call(
        _head_body,
        out_shape=jax.ShapeDtypeStruct((1, 3 * C), f32),
    )(y, w1, b1, w2, b2, w3, b3)


def kernel(x, edge_index, edge_attr, params):
    with jax.enable_x64(False):
        return _kernel_impl(x, edge_index, edge_attr, params)


def _kernel_impl(x, edge_index, edge_attr, params):
    x = x.astype(f32)
    ea = edge_attr.astype(f32)
    src = edge_index[0].astype(i32)
    dst = edge_index[1].astype(i32)

    padn = NEPAD - NE
    src_p = jnp.concatenate([src, jnp.zeros((padn,), i32)]).reshape(NW, NCHUNK, CH)
    dst_p = jnp.concatenate([dst, jnp.full((padn,), NPAD - 1, i32)]).reshape(NW, NCHUNK, CH)
    ea_p = jnp.concatenate([ea, jnp.zeros((padn, 4), f32)])
    z64 = jnp.zeros((ZROWS, C), f32)
    z16 = jnp.zeros((ZROWS, 16), f32)
    zpad = jnp.zeros((NPAD - N,), f32)

    gat, bn, attn, mlp = params["gat"], params["bn"], params["attn"], params["mlp"]
    wecat = jnp.stack([(gat[l]["W_edge"].astype(f32)
                        @ gat[l]["att_edge"][0].astype(f32)) for l in range(3)]
                      + [jnp.zeros((4,), f32)], 0)
    wec16 = jnp.concatenate([wecat, jnp.zeros((4, 12), f32)], 1)

    ae_all, sr_flat = _ae_all(ea_p, wecat.T)

    attv0 = jnp.stack([gat[0]["att_src"][0], gat[0]["att_dst"][0]]).astype(f32)
    h, av = _prep1(x, gat[0]["W"].astype(f32), attv0)

    s_parts = None
    qkv = None
    x_cur = None
    for l in range(3):
        asrc_p = jnp.concatenate([av[:, 0], zpad])
        adst_p = jnp.concatenate([av[:, 1], zpad])
        if l == 0:
            sr = sr_flat.reshape(NW, NCHUNK, CH, 16)
            p_, d_, s_ = _sc_first(src_p, dst_p, sr, asrc_p, adst_p, h,
                                   z64, z16)
            s_parts = s_
        else:
            ae_l = ae_all[:, l].reshape(NW, NCHR, CHR)
            p_, d_ = _sc_rest(src_p.reshape(NW, NCHR, CHR), dst_p, ae_l,
                              asrc_p, adst_p, h, z64, z16)
        d_ = [d_[0].reshape(NPAD, 1)[:N], d_[1].reshape(NPAD, 1)[:N]]
        b = bn[l]
        scale = (b["gamma"] / jnp.sqrt(b["var"] + f32(1e-5))).astype(f32)
        shift = (b["beta"] - b["mean"] * scale).astype(f32)
        bias = gat[l]["bias"].astype(f32)
        if l < 2:
            attv = jnp.stack([gat[l + 1]["att_src"][0],
                              gat[l + 1]["att_dst"][0]]).astype(f32)
            xo, h, av = _combine(
                l, p_, d_, s_parts, h, av, wec16[l:l + 1], bias[None, :],
                scale[None, :], shift[None, :],
                gat[l + 1]["W"].astype(f32), attv)
        else:
            x_cur, qkv = _combine(
                2, p_, d_, s_parts, h, av, wec16[l:l + 1], bias[None, :],
                scale[None, :], shift[None, :],
                attn["in_w"].astype(f32).T, attn["in_b"].astype(f32)[None, :])

    y = _mha(qkv, x_cur, attn["out_w"].astype(f32).T,
             attn["out_b"].astype(f32)[None, :])
    q = _head(y, mlp["W1"].astype(f32), mlp["b1"].astype(f32)[None, :],
              mlp["W2"].astype(f32), mlp["b2"].astype(f32)[None, :],
              mlp["W3"].astype(f32), mlp["b3"].astype(f32)[None, :])
    return q.reshape(1, 3, 64)


# speedup vs baseline: 2.1018x; 2.1018x over previous
import functools

import jax
import jax.numpy as jnp
import numpy as np
from jax import lax
from jax.experimental import pallas as pl
from jax.experimental.pallas import tpu as pltpu
from jax.experimental.pallas import tpu_sc as plsc

N = 10000
NPAD = 10240
NE = 320000
NEPAD = 327680
IN_DIM = 128
C = 64
NC, NS, LANES = 2, 16, 16
NW = NC * NS
EPW = NEPAD // NW
CH = 128
NCHUNK = EPW // CH
ZROWS = NPAD // NS
OROWS = 1000

f32 = jnp.float32
i32 = jnp.int32
_Z = np.int32(0)

_GDN = lax.GatherDimensionNumbers(offset_dims=(), collapsed_slice_dims=(0,),
                                  start_index_map=(0,))


def _bcast(v, j):
    idx = jnp.full((LANES, 1), j, i32)
    return lax.gather(v, idx, _GDN, (1,),
                      mode=lax.GatherScatterMode.PROMISE_IN_BOUNDS)


def _sc_body(first, src_h, dst_h, ae_h, asrc_h, adst_h, h_h, z64_h, z16_h,
             *rest):
    assert first
    (outp_h, outd_h, outs_h,
     acc_p, acc_d, acc_s,
     asrc_v, adst_v, srcv, dstv, rows_a, rows_b, den2, idxb, sbuf,
     sem_a, sem_b) = rest

    cid = lax.axis_index("c")
    sid = lax.axis_index("s")
    wid = sid * NC + cid
    tid = sid

    pltpu.sync_copy(asrc_h, asrc_v)
    pltpu.sync_copy(adst_h, adst_v)
    pltpu.sync_copy(src_h.at[wid], srcv)
    pltpu.sync_copy(dst_h.at[wid], dstv)

    pltpu.sync_copy(z64_h, acc_p.at[pl.ds(tid * ZROWS, ZROWS), :])
    pltpu.sync_copy(z16_h.at[pl.ds(0, ZROWS // NS), :],
                    acc_d.at[pl.ds(tid * (ZROWS // NS), ZROWS // NS), :])
    if first:
        pltpu.sync_copy(z16_h, acc_s.at[pl.ds(tid * ZROWS, ZROWS), :])
    pltpu.sync_copy(z16_h, den2)
    for j in range(ZROWS // CH):
        for g in range(CH // LANES):
            idxb[j, pl.ds(g * LANES, LANES)] = (
                lax.iota(i32, LANES) + (j * CH + g * LANES))
    plsc.subcore_barrier()

    def issue_gather(c, rows, sem):
        pltpu.async_copy(h_h.at[srcv.at[c]], rows, sem)

    def wait_gather(c, rows, sem):
        pltpu.make_async_copy(h_h.at[srcv.at[c]], rows, sem).wait()

    def do_chunk(c, rows):
        if first:
            pltpu.sync_copy(ae_h.at[wid, c], sbuf)

        def grp(g, carry2):
            sl = pl.ds(g * np.int32(LANES), LANES)
            s16 = srcv[c, sl]
            d16 = dstv[c, sl]
            a1 = plsc.load_gather(asrc_v, [s16])
            a2 = plsc.load_gather(adst_v, [d16])
            e16 = lax.iota(i32, LANES) + g * np.int32(LANES)
            if first:
                ae16 = plsc.load_gather(sbuf, [e16, jnp.full((LANES,), 5, i32)])
            else:
                ae16 = aev[c, sl]
            t = a1 + a2 + ae16
            t = jnp.maximum(t, f32(0.2) * t)
            p16 = jnp.exp(t)
            plsc.addupdate_scatter(
                den2, [lax.shift_right_logical(d16, 4),
                       lax.bitwise_and(d16, jnp.full((LANES,), 15, i32))],
                p16)
            for j in range(LANES):
                pj = _bcast(p16, j)
                e = g * np.int32(LANES) + np.int32(j)
                for v in range(C // LANES):
                    sl2 = pl.ds(v * LANES, LANES)
                    rows[e, sl2] = rows[e, sl2] * pj
            return carry2

        lax.fori_loop(np.int32(0), np.int32(CH // LANES), grp, np.int32(0))
        pltpu.sync_copy(rows, acc_p.at[dstv.at[c]], add=True)
        if first:
            pltpu.sync_copy(sbuf, acc_s.at[dstv.at[c]], add=True)

    issue_gather(np.int32(0), rows_a, sem_a)

    def pair(i, carry):
        c0 = i * np.int32(2)
        c1 = c0 + np.int32(1)
        issue_gather(c1, rows_b, sem_b)
        wait_gather(c0, rows_a, sem_a)
        do_chunk(c0, rows_a)

        @pl.when(i < np.int32(NCHUNK // 2 - 1))
        def _prefetch():
            issue_gather(c1 + np.int32(1), rows_a, sem_a)

        wait_gather(c1, rows_b, sem_b)
        do_chunk(c1, rows_b)
        return carry

    lax.fori_loop(np.int32(0), np.int32(NCHUNK // 2), pair, np.int32(0))

    for j in range(ZROWS // CH):
        pltpu.sync_copy(den2.at[pl.ds(j * CH, CH), :],
                        acc_d.at[idxb.at[j]], add=True)
    plsc.subcore_barrier()

    @pl.when(tid < N // OROWS)
    def _copyout():
        r0 = tid * np.int32(OROWS)
        pltpu.sync_copy(acc_p.at[pl.ds(r0, OROWS), :],
                        outp_h.at[cid, pl.ds(r0, OROWS), :])
        if first:
            pltpu.sync_copy(acc_s.at[pl.ds(r0, OROWS), :],
                            outs_h.at[cid, pl.ds(r0, OROWS), :])

    @pl.when(tid == np.int32(0))
    def _copyden():
        pltpu.sync_copy(acc_d, outd_h.at[cid])


CHR = 256
NCHR = EPW // CHR


def _sc_rest_body(src_h, dst_h, ae_h, asrc_h, adst_h, h_h, z64_h, z16_h,
                  outp_h, outd_h,
                  acc_p, acc_d,
                  asrc_v, adst_v, srcv, dstv, rows_a, rows_b, aeb_a, aeb_b,
                  den2, idxb, sem_a, sem_b):
    cid = lax.axis_index("c")
    sid = lax.axis_index("s")
    wid = sid * NC + cid
    tid = sid

    pltpu.sync_copy(asrc_h, asrc_v)
    pltpu.sync_copy(adst_h, adst_v)
    pltpu.sync_copy(src_h.at[wid], srcv)
    pltpu.sync_copy(dst_h.at[wid], dstv)

    pltpu.sync_copy(z64_h, acc_p.at[pl.ds(tid * ZROWS, ZROWS), :])
    pltpu.sync_copy(z16_h.at[pl.ds(0, ZROWS // NS), :],
                    acc_d.at[pl.ds(tid * (ZROWS // NS), ZROWS // NS), :])
    pltpu.sync_copy(z16_h, den2)
    for j in range(ZROWS // CH):
        for g in range(CH // LANES):
            idxb[j, pl.ds(g * LANES, LANES)] = (
                lax.iota(i32, LANES) + (j * CH + g * LANES))
    plsc.subcore_barrier()

    def issue_gather(c, rows, aeb, sem):
        pltpu.async_copy(h_h.at[srcv.at[c, pl.ds(0, CH)]],
                         rows.at[pl.ds(0, CH), :], sem)
        pltpu.async_copy(h_h.at[srcv.at[c, pl.ds(CH, CH)]],
                         rows.at[pl.ds(CH, CH), :], sem)
        pltpu.async_copy(ae_h.at[wid, c], aeb, sem)

    def wait_gather(c, rows, aeb, sem):
        pltpu.make_async_copy(h_h.at[srcv.at[c, pl.ds(0, CH)]],
                              rows.at[pl.ds(0, CH), :], sem).wait()
        pltpu.make_async_copy(h_h.at[srcv.at[c, pl.ds(CH, CH)]],
                              rows.at[pl.ds(CH, CH), :], sem).wait()
        pltpu.make_async_copy(ae_h.at[wid, c], aeb, sem).wait()

    def do_chunk(c, rows, aeb):
        for k in range(CHR // CH):
            def grp(g, carry2):
                off = g * np.int32(LANES) + np.int32(k * CH)
                s16 = srcv[c, pl.ds(off, LANES)]
                d16 = dstv[c * np.int32(2) + np.int32(k), pl.ds(g * np.int32(LANES), LANES)]
                a1 = plsc.load_gather(asrc_v, [s16])
                a2 = plsc.load_gather(adst_v, [d16])
                ae16 = aeb[pl.ds(off, LANES)]
                t = a1 + a2 + ae16
                t = jnp.maximum(t, f32(0.2) * t)
                p16 = jnp.exp(t)
                plsc.addupdate_scatter(
                    den2, [lax.shift_right_logical(d16, 4),
                           lax.bitwise_and(d16, jnp.full((LANES,), 15, i32))],
                    p16)
                for j in range(LANES):
                    pj = _bcast(p16, j)
                    e = off + np.int32(j)
                    for v in range(C // LANES):
                        sl2 = pl.ds(v * LANES, LANES)
                        rows[e, sl2] = rows[e, sl2] * pj
                return carry2

            lax.fori_loop(np.int32(0), np.int32(CH // LANES), grp, np.int32(0))
            pltpu.sync_copy(rows.at[pl.ds(k * CH, CH), :],
                            acc_p.at[dstv.at[c * np.int32(2) + np.int32(k)]],
                            add=True)

    issue_gather(np.int32(0), rows_a, aeb_a, sem_a)

    def pair(i, carry):
        c0 = i * np.int32(2)
        c1 = c0 + np.int32(1)
        issue_gather(c1, rows_b, aeb_b, sem_b)
        wait_gather(c0, rows_a, aeb_a, sem_a)
        do_chunk(c0, rows_a, aeb_a)

        @pl.when(i < np.int32(NCHR // 2 - 1))
        def _prefetch():
            issue_gather(c1 + np.int32(1), rows_a, aeb_a, sem_a)

        wait_gather(c1, rows_b, aeb_b, sem_b)
        do_chunk(c1, rows_b, aeb_b)
        return carry

    lax.fori_loop(np.int32(0), np.int32(NCHR // 2), pair, np.int32(0))

    for j in range(ZROWS // CH):
        pltpu.sync_copy(den2.at[pl.ds(j * CH, CH), :],
                        acc_d.at[idxb.at[j]], add=True)
    plsc.subcore_barrier()

    @pl.when(tid < N // OROWS)
    def _copyout():
        r0 = tid * np.int32(OROWS)
        pltpu.sync_copy(acc_p.at[pl.ds(r0, OROWS), :],
                        outp_h.at[cid, pl.ds(r0, OROWS), :])

    @pl.when(tid == np.int32(0))
    def _copyden():
        pltpu.sync_copy(acc_d, outd_h.at[cid])


def _make_sc_rest():
    mesh = plsc.VectorSubcoreMesh(core_axis_name="c", subcore_axis_name="s")
    return functools.partial(
        pl.kernel,
        out_type=[
            jax.ShapeDtypeStruct((NC, N, C), f32),
            jax.ShapeDtypeStruct((NC, ZROWS, 16), f32),
        ],
        mesh=mesh,
        scratch_types=[
            pltpu.VMEM_SHARED((NPAD, C), f32),
            pltpu.VMEM_SHARED((ZROWS, 16), f32),
            pltpu.VMEM((NPAD,), f32),
            pltpu.VMEM((NPAD,), f32),
            pltpu.VMEM((NCHR, CHR), i32),
            pltpu.VMEM((2 * NCHR, CH), i32),
            pltpu.VMEM((CHR, C), f32),
            pltpu.VMEM((CHR, C), f32),
            pltpu.VMEM((CHR,), f32),
            pltpu.VMEM((CHR,), f32),
            pltpu.VMEM((ZROWS, 16), f32),
            pltpu.VMEM((ZROWS // CH, CH), i32),
            pltpu.SemaphoreType.DMA,
            pltpu.SemaphoreType.DMA,
        ],
        compiler_params=pltpu.CompilerParams(needs_layout_passes=False,
                                             use_tc_tiling_on_sc=False),
    )(_sc_rest_body)


def _make_sc(first):
    out_type = [
        jax.ShapeDtypeStruct((NC, N, C), f32),
        jax.ShapeDtypeStruct((NC, ZROWS, 16), f32),
    ]
    scratch = [
        pltpu.VMEM_SHARED((NPAD, C), f32),
        pltpu.VMEM_SHARED((ZROWS, 16), f32),
    ]
    tile_scr = [
        pltpu.VMEM((NPAD,), f32),
        pltpu.VMEM((NPAD,), f32),
        pltpu.VMEM((NCHUNK, CH), i32),
        pltpu.VMEM((NCHUNK, CH), i32),
    ]
    if not first:
        tile_scr.append(pltpu.VMEM((NCHUNK, CH), f32))
    tile_scr += [
        pltpu.VMEM((CH, C), f32),
        pltpu.VMEM((CH, C), f32),
        pltpu.VMEM((ZROWS, 16), f32),
        pltpu.VMEM((ZROWS // CH, CH), i32),
    ]
    if first:
        out_type.append(jax.ShapeDtypeStruct((NC, N, 16), f32))
        scratch.append(pltpu.VMEM_SHARED((NPAD, 16), f32))
        tile_scr.append(pltpu.VMEM((CH, 16), f32))
    mesh = plsc.VectorSubcoreMesh(core_axis_name="c", subcore_axis_name="s")
    return functools.partial(
        pl.kernel,
        out_type=out_type,
        mesh=mesh,
        scratch_types=(scratch + tile_scr
                       + [pltpu.SemaphoreType.DMA] * 2),
        compiler_params=pltpu.CompilerParams(needs_layout_passes=False,
                                             use_tc_tiling_on_sc=False),
    )(functools.partial(_sc_body, first))


_sc_first = _make_sc(True)
_sc_rest = _make_sc_rest()


_BE = 8192


def _ae_body(ea_ref, w_ref, out_ref, sr_ref):
    ea = ea_ref[...]
    ae = jnp.dot(ea, w_ref[...], preferred_element_type=f32)
    out_ref[...] = ae
    sr_ref[...] = jnp.concatenate(
        [ea, jnp.ones((_BE, 1), f32), ae[:, 0:1], jnp.zeros((_BE, 10), f32)],
        axis=1)


def _ae_all(ea_p, wecat_t):
    return pl.pallas_call(
        _ae_body,
        grid=(NEPAD // _BE,),
        in_specs=[
            pl.BlockSpec((_BE, 4), lambda i: (i, _Z)),
            pl.BlockSpec((4, 4), lambda i: (_Z, _Z)),
        ],
        out_specs=[
            pl.BlockSpec((_BE, 4), lambda i: (i, _Z)),
            pl.BlockSpec((_BE, 16), lambda i: (i, _Z)),
        ],
        out_shape=[
            jax.ShapeDtypeStruct((NEPAD, 4), f32),
            jax.ShapeDtypeStruct((NEPAD, 16), f32),
        ],
    )(ea_p, wecat_t)


_BN = 2000


def _prep_body(x_ref, w_ref, att_ref, h_ref, av_ref):
    h = jnp.dot(x_ref[...], w_ref[...], preferred_element_type=f32)
    h_ref[...] = h
    a1 = jnp.sum(h * att_ref[0:1, :], axis=1, keepdims=True)
    a2 = jnp.sum(h * att_ref[1:2, :], axis=1, keepdims=True)
    av_ref[...] = jnp.concatenate([a1, a2], axis=1)


def _prep1(x, w, attv):
    return pl.pallas_call(
        _prep_body,
        grid=(N // _BN,),
        in_specs=[
            pl.BlockSpec((_BN, IN_DIM), lambda i: (i, _Z)),
            pl.BlockSpec((IN_DIM, C), lambda i: (_Z, _Z)),
            pl.BlockSpec((2, C), lambda i: (_Z, _Z)),
        ],
        out_specs=[
            pl.BlockSpec((_BN, C), lambda i: (i, _Z)),
            pl.BlockSpec((_BN, 2), lambda i: (i, _Z)),
        ],
        out_shape=[
            jax.ShapeDtypeStruct((N, C), f32),
            jax.ShapeDtypeStruct((N, 2), f32),
        ],
    )(x, w, attv)


def _combine_body(mode, p0, p1, d0, d1, s0, s1, h_ref, av_ref, wec, bias,
                  scale, shift, wn, attn_or_b, xo_ref, o1_ref, o2_ref=None):
    num = p0[...] + p1[...]
    den = d0[:, 0:1] + d1[:, 0:1]
    s = s0[...] + s1[...]
    mean_ea = s[:, 0:4] / jnp.maximum(s[:, 4:5], f32(1.0))
    ae_self = jnp.sum(mean_ea * wec[0:1, 0:4], axis=1, keepdims=True)
    ts = av_ref[:, 0:1] + av_ref[:, 1:2] + ae_self
    ts = jnp.maximum(ts, f32(0.2) * ts)
    ps = jnp.exp(ts)
    xo = (num + ps * h_ref[...]) / (den + ps + f32(1e-16)) + bias[0:1, :]
    xo = jnp.maximum(xo * scale[0:1, :] + shift[0:1, :], f32(0.0))
    xo_ref[...] = xo
    if mode < 2:
        hn = jnp.dot(xo, wn[...], preferred_element_type=f32)
        o1_ref[...] = hn
        a1 = jnp.sum(hn * attn_or_b[0:1, :], axis=1, keepdims=True)
        a2 = jnp.sum(hn * attn_or_b[1:2, :], axis=1, keepdims=True)
        o2_ref[...] = jnp.concatenate([a1, a2], axis=1)
    else:
        o1_ref[...] = (jnp.dot(xo, wn[...], preferred_element_type=f32)
                       + attn_or_b[0:1, :])


def _combine(mode, p, d, s, h, av, wec, bias, scale, shift, wn, attn_or_b):
    bcast = lambda i: (_Z, _Z)
    row64 = pl.BlockSpec((_BN, C), lambda i: (i, _Z))
    row16 = pl.BlockSpec((_BN, 16), lambda i: (i, _Z))
    row1 = pl.BlockSpec((_BN, 1), lambda i: (i, _Z))
    in_specs = [
        row64, row64, row1, row1, row16, row16, row64,
        pl.BlockSpec((_BN, 2), lambda i: (i, _Z)),
        pl.BlockSpec((1, 16), bcast),
        pl.BlockSpec((1, C), bcast),
        pl.BlockSpec((1, C), bcast),
        pl.BlockSpec((1, C), bcast),
    ]
    if mode < 2:
        in_specs += [pl.BlockSpec((C, C), bcast), pl.BlockSpec((2, C), bcast)]
        out_specs = [row64, row64, pl.BlockSpec((_BN, 2), lambda i: (i, _Z))]
        out_shape = [
            jax.ShapeDtypeStruct((N, C), f32),
            jax.ShapeDtypeStruct((N, C), f32),
            jax.ShapeDtypeStruct((N, 2), f32),
        ]
    else:
        in_specs += [pl.BlockSpec((C, 3 * C), bcast),
                     pl.BlockSpec((1, 3 * C), bcast)]
        out_specs = [row64, pl.BlockSpec((_BN, 3 * C), lambda i: (i, _Z))]
        out_shape = [
            jax.ShapeDtypeStruct((N, C), f32),
            jax.ShapeDtypeStruct((N, 3 * C), f32),
        ]
    return pl.pallas_call(
        functools.partial(_combine_body, mode),
        grid=(N // _BN,),
        in_specs=in_specs,
        out_specs=out_specs,
        out_shape=out_shape,
    )(p[0], p[1], d[0], d[1], s[0], s[1], h, av, wec, bias, scale, shift,
      wn, attn_or_b)


_BQ = 1000


def _mha_body(qkv_ref, qkvf_ref, x_ref, outw_ref, outb_ref, y_ref):
    q = qkv_ref[:, 0:C]
    nkb = N // _BQ

    qb16 = q.astype(jnp.bfloat16)

    def kblk(j, carry):
        num, den = carry
        jb = j * np.int32(_BQ)
        kb = qkvf_ref[pl.ds(jb, _BQ), C:2 * C].astype(jnp.bfloat16)
        vb = qkvf_ref[pl.ds(jb, _BQ), 2 * C:3 * C].astype(jnp.bfloat16)
        sc = lax.dot_general(qb16, kb, (((1,), (1,)), ((), ())),
                             preferred_element_type=f32) * f32(0.125)
        e = jnp.exp(sc)
        den = den + jnp.sum(e, axis=1, keepdims=True)
        num = num + jnp.dot(e.astype(jnp.bfloat16), vb,
                            preferred_element_type=f32)
        return num, den

    num0 = jnp.zeros((_BQ, C), f32)
    den0 = jnp.zeros((_BQ, 1), f32)
    num, den = lax.fori_loop(np.int32(0), np.int32(nkb), kblk, (num0, den0))
    y = num / den
    y_ref[...] = (jnp.dot(y, outw_ref[...], preferred_element_type=f32)
                  + outb_ref[0:1, :] + x_ref[...])


def _mha(qkv, x3, outw_t, outb):
    return pl.pallas_call(
        _mha_body,
        grid=(N // _BQ,),
        in_specs=[
            pl.BlockSpec((_BQ, 3 * C), lambda i: (i, _Z)),
            pl.BlockSpec((N, 3 * C), lambda i: (_Z, _Z)),
            pl.BlockSpec((_BQ, C), lambda i: (i, _Z)),
            pl.BlockSpec((C, C), lambda i: (_Z, _Z)),
            pl.BlockSpec((1, C), lambda i: (_Z, _Z)),
        ],
        out_specs=pl.BlockSpec((_BQ, C), lambda i: (i, _Z)),
        out_shape=jax.ShapeDtypeStruct((N, C), f32),
    )(qkv, qkv, x3, outw_t, outb)


def _head_body(y_ref, w1, b1, w2, b2, w3, b3, out_ref):
    y = y_ref[...]
    g = jnp.concatenate([jnp.mean(y, axis=0, keepdims=True),
                         jnp.max(y, axis=0, keepdims=True)], axis=1)
    h1 = jnp.maximum(jnp.dot(g, w1[...], preferred_element_type=f32)
                     + b1[0:1, :], f32(0.0))
    h2 = jnp.maximum(jnp.dot(h1, w2[...], preferred_element_type=f32)
                     + b2[0:1, :], f32(0.0))
    out_ref[...] = (jnp.dot(h2, w3[...], preferred_element_type=f32)
                    + b3[0:1, :])


def _head(y, w1, b1, w2, b2, w3, b3):
    return pl.pallas_call(
        _head_body,
        out_shape=jax.ShapeDtypeStruct((1, 3 * C), f32),
    )(y, w1, b1, w2, b2, w3, b3)


def kernel(x, edge_index, edge_attr, params):
    with jax.enable_x64(False):
        return _kernel_impl(x, edge_index, edge_attr, params)


def _kernel_impl(x, edge_index, edge_attr, params):
    x = x.astype(f32)
    ea = edge_attr.astype(f32)
    src = edge_index[0].astype(i32)
    dst = edge_index[1].astype(i32)

    padn = NEPAD - NE
    src_p = jnp.concatenate([src, jnp.zeros((padn,), i32)]).reshape(NW, NCHUNK, CH)
    dst_p = jnp.concatenate([dst, jnp.full((padn,), NPAD - 1, i32)]).reshape(NW, NCHUNK, CH)
    ea_p = jnp.concatenate([ea, jnp.zeros((padn, 4), f32)])
    z64 = jnp.zeros((ZROWS, C), f32)
    z16 = jnp.zeros((ZROWS, 16), f32)
    zpad = jnp.zeros((NPAD - N,), f32)

    gat, bn, attn, mlp = params["gat"], params["bn"], params["attn"], params["mlp"]
    wecat = jnp.stack([(gat[l]["W_edge"].astype(f32)
                        @ gat[l]["att_edge"][0].astype(f32)) for l in range(3)]
                      + [jnp.zeros((4,), f32)], 0)
    wec16 = jnp.concatenate([wecat, jnp.zeros((4, 12), f32)], 1)

    ae_all, sr_flat = _ae_all(ea_p, wecat.T)

    attv0 = jnp.stack([gat[0]["att_src"][0], gat[0]["att_dst"][0]]).astype(f32)
    h, av = _prep1(x, gat[0]["W"].astype(f32), attv0)

    s_parts = None
    qkv = None
    x_cur = None
    for l in range(3):
        asrc_p = jnp.concatenate([av[:, 0], zpad])
        adst_p = jnp.concatenate([av[:, 1], zpad])
        if l == 0:
            sr = sr_flat.reshape(NW, NCHUNK, CH, 16)
            p_, d_, s_ = _sc_first(src_p, dst_p, sr, asrc_p, adst_p, h,
                                   z64, z16)
            s_parts = s_
        else:
            ae_l = ae_all[:, l].reshape(NW, NCHR, CHR)
            p_, d_ = _sc_rest(src_p.reshape(NW, NCHR, CHR), dst_p, ae_l,
                              asrc_p, adst_p, h, z64, z16)
        d_ = [d_[0].reshape(NPAD, 1)[:N], d_[1].reshape(NPAD, 1)[:N]]
        b = bn[l]
        scale = (b["gamma"] / jnp.sqrt(b["var"] + f32(1e-5))).astype(f32)
        shift = (b["beta"] - b["mean"] * scale).astype(f32)
        bias = gat[l]["bias"].astype(f32)
        if l < 2:
            attv = jnp.stack([gat[l + 1]["att_src"][0],
                              gat[l + 1]["att_dst"][0]]).astype(f32)
            xo, h, av = _combine(
                l, p_, d_, s_parts, h, av, wec16[l:l + 1], bias[None, :],
                scale[None, :], shift[None, :],
                gat[l + 1]["W"].astype(f32), attv)
        else:
            x_cur, qkv = _combine(
                2, p_, d_, s_parts, h, av, wec16[l:l + 1], bias[None, :],
                scale[None, :], shift[None, :],
                attn["in_w"].astype(f32).T, attn["in_b"].astype(f32)[None, :])

    y = _mha(qkv, x_cur, attn["out_w"].astype(f32).T,
             attn["out_b"].astype(f32)[None, :])
    q = _head(y, mlp["W1"].astype(f32), mlp["b1"].astype(f32)[None, :],
              mlp["W2"].astype(f32), mlp["b2"].astype(f32)[None, :],
              mlp["W3"].astype(f32), mlp["b3"].astype(f32)[None, :])
    return q.reshape(1, 3, 64)
